# baseline (speedup 1.0000x reference)
import functools

import jax
import jax.numpy as jnp
from jax import lax
from jax.experimental import pallas as pl
from jax.experimental.pallas import tpu as pltpu
from jax.experimental.pallas import tpu_sc as plsc

_NC = 2
_NS = 16
_NW = _NC * _NS
_L = 16


def _sc_body(CH, B, N, C, k, pts_hbm, idx_hbm, out_hbm,
             idx_v, rid, rows, stag, sem):
    PPT = N * B // _NW
    KC = k * CH
    n_chunks = PPT // CH

    wid = lax.axis_index("s") * _NC + lax.axis_index("c")
    base_pt = wid * PPT
    b = base_pt // N
    boff2 = b * (N // 2)
    n_base = base_pt - b * N

    io = lax.iota(jnp.int32, _L)
    ccol = [io + cc * _L for cc in range(C // _L)]

    pltpu.sync_copy(idx_hbm.at[wid], idx_v)

    def chunk_body(ch, _):
        q0 = ch * KC
        for s in range(KC // _L):
            v = idx_v[pl.ds(q0 + s * _L, _L)]
            rid[pl.ds(s * _L, _L)] = (v >> 1) + boff2
        p0h = boff2 + (n_base + ch * CH) // 2
        tail = jnp.where(io < CH // 2, io + p0h, p0h)
        rid[pl.ds(KC, _L)] = tail
        copy = pltpu.async_copy(pts_hbm.at[rid], rows, sem)
        copy.wait()

        def point_group(p):
            xr = [rows[KC + (p // 2), pl.ds((p % 2) * C + cc * _L, _L)]
                  for cc in range(C // _L)]
            pvec = lax.broadcast(p, (_L,))
            pv_a = idx_v[pl.ds(q0 + p * k, _L)] & 1
            pv_b = idx_v[pl.ds(q0 + p * k + (k - _L), _L)] & 1
            for j in range(k):
                par = pv_a[j] if j < _L else pv_b[j - (k - _L)]
                parvec = lax.broadcast(par, (_L,)) > 0
                jvec = lax.broadcast(j, (_L,))
                for cc in range(C // _L):
                    lo = rows[p * k + j, pl.ds(cc * _L, _L)]
                    hi = rows[p * k + j, pl.ds(C + cc * _L, _L)]
                    g = jnp.where(parvec, hi, lo)
                    plsc.store_scatter(stag, [pvec, ccol[cc], jvec],
                                       g - xr[cc])
                    plsc.store_scatter(stag, [pvec, ccol[cc] + C, jvec],
                                       xr[cc])

        for p in range(CH):
            point_group(p)
        n0 = n_base + ch * CH
        pltpu.sync_copy(stag, out_hbm.at[b, pl.ds(n0, CH)])
        return ()

    lax.fori_loop(0, n_chunks, chunk_body, (), unroll=False)


def kernel(points, idx):
    B, N, C = points.shape
    k = idx.shape[2]
    BN = B * N
    CH = 4

    pts2 = points.reshape(BN // 2, 2 * C)
    idx_t = idx.reshape(_NW, (BN // _NW) * k)

    mesh = plsc.VectorSubcoreMesh(core_axis_name="c", subcore_axis_name="s")
    body = functools.partial(_sc_body, CH, B, N, C, k)
    sc_fn = pl.kernel(
        body,
        out_type=jax.ShapeDtypeStruct((B, N, 2 * C, k), jnp.float32),
        mesh=mesh,
        compiler_params=pltpu.CompilerParams(needs_layout_passes=False,
                                             use_tc_tiling_on_sc=True),
        scratch_types=[
            pltpu.VMEM(((BN // _NW) * k,), jnp.int32),
            pltpu.VMEM((k * CH + _L,), jnp.int32),
            pltpu.VMEM((k * CH + _L, 2 * C), jnp.float32),
            pltpu.VMEM((CH, 2 * C, k), jnp.float32),
            pltpu.SemaphoreType.DMA,
        ],
    )
    return sc_fn(pts2, idx_t)


# speedup vs baseline: 1.4206x; 1.4206x over previous
import functools

import jax
import jax.numpy as jnp
from jax import lax
from jax.experimental import pallas as pl
from jax.experimental.pallas import tpu as pltpu
from jax.experimental.pallas import tpu_sc as plsc

_NC = 2
_NS = 16
_NW = _NC * _NS
_L = 16


def _sc_body(CH, B, N, C, k, pts_hbm, idx_hbm, out_hbm,
             idx_v, rid0, rid1, rows0, rows1, stag0, stag1,
             gs0, gs1, os0, os1):
    PPT = N * B // _NW
    KC = k * CH
    NR = KC + 8
    n_chunks = PPT // CH
    rids = (rid0, rid1)
    rowss = (rows0, rows1)
    stags = (stag0, stag1)
    gsems = (gs0, gs1)
    osems = (os0, os1)

    wid = lax.axis_index("s") * _NC + lax.axis_index("c")
    base_pt = wid * PPT
    b = base_pt // N
    boff2 = b * (N // 2)
    n_base = base_pt - b * N

    io = lax.iota(jnp.int32, _L)
    ccol = [io + cc * _L for cc in range(C // _L)]
    ccol2 = [c_ + C for c_ in ccol]

    pltpu.sync_copy(idx_hbm.at[wid], idx_v)

    def prep_rid(ch, q):
        q0 = ch * KC
        for s in range(KC // _L):
            v = idx_v[pl.ds(q0 + s * _L, _L)]
            rids[q][pl.ds(s * _L, _L)] = (v >> 1) + boff2
        rem = KC - (KC // _L) * _L
        vt = (idx_v[pl.ds(q0 + KC - rem, _L)] >> 1) + boff2
        p0h = boff2 + (n_base + ch * CH) // 2
        tail = jnp.where(io < rem, vt, p0h)
        rids[q][pl.ds(KC - rem, _L)] = tail

    def start_gather(q):
        return pltpu.async_copy(pts_hbm.at[rids[q]], rowss[q], gsems[q])

    def wait_gather(q):
        pltpu.make_async_copy(pts_hbm.at[rids[q]], rowss[q], gsems[q]).wait()

    def out_slice(ch):
        return out_hbm.at[b, pl.ds(n_base + ch * CH, CH)]

    def compute(ch, q):
        q0 = ch * KC
        rows = rowss[q]
        stag = stags[q]
        for p in range(CH):
            xr = [rows[KC, pl.ds(p * C + cc * _L, _L)]
                  for cc in range(C // _L)]
            pvec = lax.broadcast(p, (_L,))
            pv_a = idx_v[pl.ds(q0 + p * k, _L)] & 1
            pv_b = idx_v[pl.ds(q0 + p * k + (k - _L), _L)] & 1
            for j in range(k):
                par = pv_a[j] if j < _L else pv_b[j - (k - _L)]
                jvec = lax.broadcast(j, (_L,))
                for cc in range(C // _L):
                    g = rows[p * k + j, pl.ds(par * C + cc * _L, _L)]
                    plsc.store_scatter(stag, [pvec, ccol[cc], jvec],
                                       g - xr[cc])
                    plsc.store_scatter(stag, [pvec, ccol2[cc], jvec],
                                       xr[cc])

    n_half = n_chunks // 2
    prep_rid(0, 0)
    start_gather(0)

    def pair_body(i, _):
        ch0 = i * 2
        for q in range(2):
            ch = ch0 + q
            if q == 0:
                prep_rid(ch + 1, 1)
                start_gather(1)
            else:
                @pl.when(i < n_half - 1)
                def _():
                    prep_rid(ch + 1, 0)
                    start_gather(0)
            wait_gather(q)
            @pl.when(i > 0)
            def _():
                pltpu.make_async_copy(stags[q], out_slice(ch), osems[q]).wait()
            compute(ch, q)
            pltpu.async_copy(stags[q], out_slice(ch), osems[q])
        return ()

    lax.fori_loop(0, n_half, pair_body, (), unroll=False)
    for q, ch in ((0, n_chunks - 2), (1, n_chunks - 1)):
        pltpu.make_async_copy(stags[q], out_slice(ch), osems[q]).wait()


def kernel(points, idx):
    B, N, C = points.shape
    k = idx.shape[2]
    BN = B * N
    CH = 2

    pts2 = points.reshape(BN // 2, 2 * C)
    idx_t = idx.reshape(_NW, (BN // _NW) * k)

    mesh = plsc.VectorSubcoreMesh(core_axis_name="c", subcore_axis_name="s")
    body = functools.partial(_sc_body, CH, B, N, C, k)
    NR = k * CH + 8
    sc_fn = pl.kernel(
        body,
        out_type=jax.ShapeDtypeStruct((B, N, 2 * C, k), jnp.float32),
        mesh=mesh,
        compiler_params=pltpu.CompilerParams(needs_layout_passes=False,
                                             use_tc_tiling_on_sc=True),
        scratch_types=[
            pltpu.VMEM(((BN // _NW) * k,), jnp.int32),
            pltpu.VMEM((NR,), jnp.int32),
            pltpu.VMEM((NR,), jnp.int32),
            pltpu.VMEM((NR, 2 * C), jnp.float32),
            pltpu.VMEM((NR, 2 * C), jnp.float32),
            pltpu.VMEM((CH, 2 * C, k), jnp.float32),
            pltpu.VMEM((CH, 2 * C, k), jnp.float32),
            pltpu.SemaphoreType.DMA,
            pltpu.SemaphoreType.DMA,
            pltpu.SemaphoreType.DMA,
            pltpu.SemaphoreType.DMA,
        ],
    )
    return sc_fn(pts2, idx_t)
